# baseline (speedup 1.0000x reference)
import functools

import jax
import jax.numpy as jnp
from jax import lax
from jax.experimental import pallas as pl
from jax.experimental.pallas import tpu as pltpu
from jax.experimental.pallas import tpu_sc as plsc

EMBED = 256
NH = 8
NL = 4
NP = 8
DH = 32
_SS = ((92, 160), (46, 80), (23, 40), (12, 20))
NV = 19560
_PLVL = []
_PBASE = []
_acc = 0
for _h, _w in _SS:
    _PBASE.append(_acc)
    _PLVL.append((_h + 1) * (_w + 1))
    _acc += (_h + 1) * (_w + 1)
PTOT = _acc

BS = 2
NQ = 2048
ROWS = BS * NQ * NH
PPR = NL * NP
WPR = PPR * 4
NWORK = 32
RPW = ROWS // NWORK
BLK = 8
NBLK = RPW // BLK
LANES = 16


def _mm_bias(x, w, b, block_m=512):
    M, K = x.shape
    N = w.shape[1]
    Mp = ((M + block_m - 1) // block_m) * block_m
    xp = jnp.pad(x, ((0, Mp - M), (0, 0))) if Mp != M else x

    def body(x_ref, w_ref, b_ref, o_ref):
        o_ref[...] = jnp.dot(x_ref[...], w_ref[...],
                             preferred_element_type=jnp.float32) + b_ref[...]

    out = pl.pallas_call(
        body,
        grid=(Mp // block_m,),
        in_specs=[pl.BlockSpec((block_m, K), lambda i: (i, 0)),
                  pl.BlockSpec((K, N), lambda i: (0, 0)),
                  pl.BlockSpec((1, N), lambda i: (0, 0))],
        out_specs=pl.BlockSpec((block_m, N), lambda i: (i, 0)),
        out_shape=jax.ShapeDtypeStruct((Mp, N), jnp.float32),
    )(xp, w, b[None, :])
    return out[:M]


def _quad_table(vl, H, W):
    bs = vl.shape[0]

    def body(top_ref, bot_ref, o_ref):
        a = pl.program_id(1)
        t = top_ref[0, 0] * jnp.where(a > 0, 1.0, 0.0)
        bo = bot_ref[0, 0] * jnp.where(a < H, 1.0, 0.0)
        z = jnp.zeros((1, NH * DH), jnp.float32)
        tdn = jnp.concatenate([z, t], axis=0)
        tup = jnp.concatenate([t, z], axis=0)
        bdn = jnp.concatenate([z, bo], axis=0)
        bup = jnp.concatenate([bo, z], axis=0)
        for h in range(NH):
            for s, slot in enumerate((tdn, tup, bdn, bup)):
                o_ref[0, 0, :, h, pl.ds(s * DH, DH)] = \
                    slot[:, h * DH:(h + 1) * DH]

    out = pl.pallas_call(
        body,
        grid=(bs, H + 1),
        in_specs=[
            pl.BlockSpec((1, 1, W, NH * DH),
                         lambda b, a: (b, jnp.maximum(a - 1, 0), 0, 0)),
            pl.BlockSpec((1, 1, W, NH * DH),
                         lambda b, a: (b, jnp.minimum(a, H - 1), 0, 0)),
        ],
        out_specs=pl.BlockSpec((1, 1, W + 1, NH, 4 * DH),
                               lambda b, a: (b, a, 0, 0, 0)),
        out_shape=jax.ShapeDtypeStruct((bs, H + 1, W + 1, NH, 4 * DH),
                                       jnp.float32),
    )(vl, vl)
    return out.reshape(bs * (H + 1) * (W + 1) * NH, 4 * DH)


def _sc_gather_reduce(tables, idx, wts):
    mesh = plsc.VectorSubcoreMesh(core_axis_name="c", subcore_axis_name="s")

    @functools.partial(
        pl.kernel,
        out_type=jax.ShapeDtypeStruct((ROWS * DH,), jnp.float32),
        mesh=mesh,
        scratch_types=[
            pltpu.VMEM((BLK * PPR,), jnp.int32),
            pltpu.VMEM((BLK * WPR,), jnp.float32),
            pltpu.VMEM((BLK, PPR, 128), jnp.float32),
            pltpu.VMEM((BLK * DH,), jnp.float32),
            pltpu.SemaphoreType.DMA,
        ],
    )
    def body(t0, t1, t2, t3, idx_hbm, w_hbm, out_hbm,
             idx_v, w_v, rows_v, outb, sem):
        tbls = (t0, t1, t2, t3)
        wid = lax.axis_index("s") * 2 + lax.axis_index("c")
        base = wid * RPW
        splats = [jnp.full((LANES, 1), t, jnp.int32) for t in range(LANES)]
        gdn = lax.GatherDimensionNumbers(
            offset_dims=(), collapsed_slice_dims=(0,), start_index_map=(0,))

        def bcast(vec, t):
            return lax.gather(vec, splats[t], gdn, (1,),
                              mode=lax.GatherScatterMode.PROMISE_IN_BOUNDS)

        def blk_body(i, carry):
            r0 = base + i * BLK
            pltpu.sync_copy(idx_hbm.at[pl.ds(r0 * PPR, BLK * PPR)], idx_v)
            pltpu.sync_copy(w_hbm.at[pl.ds(r0 * WPR, BLK * WPR)], w_v)
            copies = [
                pltpu.async_copy(
                    tbls[l].at[idx_v.at[pl.ds(r * PPR + l * NP, NP)]],
                    rows_v.at[r, pl.ds(l * NP, NP)], sem)
                for r in range(BLK) for l in range(NL)
            ]
            for cp in copies:
                cp.wait()

            def row_body(r, carry2):
                def chunk(cc, acc):
                    a0, a1 = acc
                    wchunk = w_v[pl.ds(r * WPR + cc * LANES, LANES)]
                    for t in range(LANES):
                        p = cc * 4 + t // 4
                        c4 = t % 4
                        wb = bcast(wchunk, t)
                        lo = rows_v[r, p, pl.ds(c4 * DH, LANES)]
                        hi = rows_v[r, p, pl.ds(c4 * DH + LANES, LANES)]
                        a0 = a0 + wb * lo
                        a1 = a1 + wb * hi
                    return a0, a1

                z = jnp.zeros((LANES,), jnp.float32)
                a0, a1 = lax.fori_loop(0, WPR // LANES, chunk, (z, z))
                outb[pl.ds(r * DH, LANES)] = a0
                outb[pl.ds(r * DH + LANES, LANES)] = a1
                return carry2

            lax.fori_loop(0, BLK, row_body, 0)
            pltpu.sync_copy(outb, out_hbm.at[pl.ds(r0 * DH, BLK * DH)])
            return carry

        lax.fori_loop(0, NBLK, blk_body, 0)

    return body(*tables, idx, wts)


def kernel(query, value, reference_points, spatial_shapes, level_start_index,
           W_off, b_off, W_attn, b_attn, W_val, b_val, W_out, b_out):
    bs, nq, d = query.shape
    nv = value.shape[1]

    v = _mm_bias(value.reshape(bs * nv, d), W_val, b_val)
    v3 = v.reshape(bs, nv, NH * DH)
    tables = []
    start = 0
    for (H, W) in _SS:
        vl = v3[:, start:start + H * W].reshape(bs, H, W, NH * DH)
        start += H * W
        tables.append(_quad_table(vl, H, W))

    qw = jnp.concatenate([W_off, W_attn], axis=1)
    qb = jnp.concatenate([b_off, b_attn], axis=0)
    qproj = _mm_bias(query.reshape(bs * nq, d), qw, qb)
    off = qproj[:, :NH * NL * NP * 2].reshape(bs, nq, NH, NL, NP, 2)
    aw = jax.nn.softmax(
        qproj[:, NH * NL * NP * 2:].reshape(bs, nq, NH, NL * NP), axis=-1)
    aw = aw.reshape(bs, nq, NH, NL, NP)

    ss_f = spatial_shapes.astype(jnp.float32)
    norm = jnp.stack([ss_f[:, 1], ss_f[:, 0]], axis=-1)
    nZ = reference_points.shape[2]
    ref = reference_points[:, :, None, None, None, :, :]
    off_n = off / norm[None, None, None, :, None, :]
    off_n = off_n.reshape(bs, nq, NH, NL, NP // nZ, nZ, 2)
    loc = (ref + off_n).reshape(bs, nq, NH, NL, NP, 2)

    Wl = jnp.array([s[1] for s in _SS], jnp.float32)[:, None]
    Hl = jnp.array([s[0] for s in _SS], jnp.float32)[:, None]
    x = loc[..., 0] * Wl - 0.5
    y = loc[..., 1] * Hl - 0.5
    x0 = jnp.floor(x)
    y0 = jnp.floor(y)
    tx = x - x0
    ty = y - y0
    xi = jnp.clip(x0, -1.0, Wl - 1.0).astype(jnp.int32)
    yi = jnp.clip(y0, -1.0, Hl - 1.0).astype(jnp.int32)
    Wp1 = jnp.array([s[1] + 1 for s in _SS], jnp.int32)[:, None]
    Hp1 = jnp.array([s[0] + 1 for s in _SS], jnp.int32)[:, None]
    barr = jnp.arange(bs, dtype=jnp.int32)[:, None, None, None, None]
    harr = jnp.arange(NH, dtype=jnp.int32)[None, None, :, None, None]
    pidx = ((barr * Hp1 + (yi + 1)) * Wp1 + (xi + 1)) * NH + harr

    vx0 = ((x0 >= 0) & (x0 < Wl)).astype(jnp.float32)
    vx1 = ((x0 + 1 >= 0) & (x0 + 1 < Wl)).astype(jnp.float32)
    vy0 = ((y0 >= 0) & (y0 < Hl)).astype(jnp.float32)
    vy1 = ((y0 + 1 >= 0) & (y0 + 1 < Hl)).astype(jnp.float32)
    wx0 = (1.0 - tx) * vx0
    wx1 = tx * vx1
    wy0 = (1.0 - ty) * vy0
    wy1 = ty * vy1
    w4 = jnp.stack([wx0 * wy0, wx1 * wy0, wx0 * wy1, wx1 * wy1], axis=-1)
    w4 = w4 * aw[..., None]

    idx = pidx.reshape(ROWS * PPR)
    wts = w4.reshape(ROWS * WPR)

    res = _sc_gather_reduce(tables, idx, wts)

    out = _mm_bias(res.reshape(bs * nq, d), W_out, b_out)
    return out.reshape(bs, nq, d)


# speedup vs baseline: 2.1555x; 2.1555x over previous
import functools

import numpy as np

import jax
import jax.numpy as jnp
from jax import lax
from jax.experimental import pallas as pl
from jax.experimental.pallas import tpu as pltpu
from jax.experimental.pallas import tpu_sc as plsc

EMBED = 256
NH = 8
NL = 4
NP = 8
DH = 32
_SS = ((92, 160), (46, 80), (23, 40), (12, 20))

BS = 2
NQ = 2048
NQT = BS * NQ
ROWS = NQT * NH
PPR = NL * NP
NWORK = 32
QPW = NQT // NWORK
QBLK = 256
LANES = 16

_lane = np.arange(NH * NL * NP)
_lane_l = (_lane // NP) % NL
_lane_h = _lane // (NL * NP)
_W_LANE = np.array([_SS[l][1] for l in _lane_l], np.float32)[None]
_H_LANE = np.array([_SS[l][0] for l in _lane_l], np.float32)[None]
_WP1_LANE = (_W_LANE + 1.0).astype(np.int32)
_HP1_LANE = (_H_LANE + 1.0).astype(np.int32)
_HL_LANE = _lane_h.astype(np.int32)[None]


def _mm_bias(x, w, b, block_m=512):
    M, K = x.shape
    N = w.shape[1]
    Mp = ((M + block_m - 1) // block_m) * block_m
    xp = jnp.pad(x, ((0, Mp - M), (0, 0))) if Mp != M else x

    def body(x_ref, w_ref, b_ref, o_ref):
        o_ref[...] = jnp.dot(x_ref[...], w_ref[...],
                             preferred_element_type=jnp.float32) + b_ref[...]

    out = pl.pallas_call(
        body,
        grid=(Mp // block_m,),
        in_specs=[pl.BlockSpec((block_m, K), lambda i: (i, 0)),
                  pl.BlockSpec((K, N), lambda i: (0, 0)),
                  pl.BlockSpec((1, N), lambda i: (0, 0))],
        out_specs=pl.BlockSpec((block_m, N), lambda i: (i, 0)),
        out_shape=jax.ShapeDtypeStruct((Mp, N), jnp.float32),
    )(xp, w, b[None, :])
    return out[:M]


def _quad_table(vl, H, W):
    bs = vl.shape[0]

    def body(top_ref, bot_ref, o_ref):
        a = pl.program_id(1)
        t = top_ref[0, 0] * jnp.where(a > 0, 1.0, 0.0)
        bo = bot_ref[0, 0] * jnp.where(a < H, 1.0, 0.0)
        z = jnp.zeros((1, NH * DH), jnp.float32)
        tdn = jnp.concatenate([z, t], axis=0)
        tup = jnp.concatenate([t, z], axis=0)
        bdn = jnp.concatenate([z, bo], axis=0)
        bup = jnp.concatenate([bo, z], axis=0)
        for h in range(NH):
            for s, slot in enumerate((tdn, tup, bdn, bup)):
                o_ref[0, 0, :, h, pl.ds(s * DH, DH)] = \
                    slot[:, h * DH:(h + 1) * DH]

    out = pl.pallas_call(
        body,
        grid=(bs, H + 1),
        in_specs=[
            pl.BlockSpec((1, 1, W, NH * DH),
                         lambda b, a: (b, jnp.maximum(a - 1, 0), 0, 0)),
            pl.BlockSpec((1, 1, W, NH * DH),
                         lambda b, a: (b, jnp.minimum(a, H - 1), 0, 0)),
        ],
        out_specs=pl.BlockSpec((1, 1, W + 1, NH, 4 * DH),
                               lambda b, a: (b, a, 0, 0, 0)),
        out_shape=jax.ShapeDtypeStruct((bs, H + 1, W + 1, NH, 4 * DH),
                                       jnp.float32),
    )(vl, vl)
    return out.reshape(bs * (H + 1) * (W + 1) * NH, 4 * DH)


def _sampling_prep(query, rpx, rpy, qw, qb):
    def body(q_ref, rx_ref, ry_ref, w_ref, b_ref, wl_ref, hl_ref,
             wp1_ref, hp1_ref, hid_ref, idx_ref, wts_ref):
        i = pl.program_id(0)
        b = i // (NQ // QBLK)
        proj = jnp.dot(q_ref[...], w_ref[...],
                       preferred_element_type=jnp.float32) + b_ref[...]
        offx = proj[:, 0:256]
        offy = proj[:, 256:512]
        logit = proj[:, 512:768]
        segs = []
        for h in range(NH):
            seg = logit[:, h * 32:(h + 1) * 32]
            m = jnp.max(seg, axis=1, keepdims=True)
            e = jnp.exp(seg - m)
            segs.append(e / jnp.sum(e, axis=1, keepdims=True))
        aw = jnp.concatenate(segs, axis=1)

        rx = jnp.tile(rx_ref[...], (1, 64))
        ry = jnp.tile(ry_ref[...], (1, 64))
        Wl = wl_ref[...]
        Hl = hl_ref[...]
        x = rx * Wl + offx - 0.5
        y = ry * Hl + offy - 0.5
        x0 = jnp.floor(x)
        y0 = jnp.floor(y)
        tx = x - x0
        ty = y - y0
        vx0 = ((x0 >= 0.0) & (x0 < Wl)).astype(jnp.float32)
        vx1 = ((x0 + 1.0 >= 0.0) & (x0 + 1.0 < Wl)).astype(jnp.float32)
        vy0 = ((y0 >= 0.0) & (y0 < Hl)).astype(jnp.float32)
        vy1 = ((y0 + 1.0 >= 0.0) & (y0 + 1.0 < Hl)).astype(jnp.float32)
        xi = jnp.clip(x0, -1.0, Wl - 1.0).astype(jnp.int32)
        yi = jnp.clip(y0, -1.0, Hl - 1.0).astype(jnp.int32)
        pidx = ((b * hp1_ref[...] + (yi + 1))
                * wp1_ref[...] + (xi + 1)) * NH + hid_ref[...]
        idx_ref[...] = pidx
        wx0 = (1.0 - tx) * vx0
        wx1 = tx * vx1
        wy0 = (1.0 - ty) * vy0
        wy1 = ty * vy1
        for c, wc in enumerate((wx0 * wy0, wx1 * wy0, wx0 * wy1, wx1 * wy1)):
            wts_ref[:, pl.ds(c * 256, 256)] = wc * aw

    idx, wts = pl.pallas_call(
        body,
        grid=(NQT // QBLK,),
        in_specs=[pl.BlockSpec((QBLK, EMBED), lambda i: (i, 0)),
                  pl.BlockSpec((QBLK, 4), lambda i: (i, 0)),
                  pl.BlockSpec((QBLK, 4), lambda i: (i, 0)),
                  pl.BlockSpec((EMBED, 768), lambda i: (0, 0)),
                  pl.BlockSpec((1, 768), lambda i: (0, 0))]
        + [pl.BlockSpec((1, 256), lambda i: (0, 0))] * 5,
        out_specs=[pl.BlockSpec((QBLK, 256), lambda i: (i, 0)),
                   pl.BlockSpec((QBLK, 1024), lambda i: (i, 0))],
        out_shape=[jax.ShapeDtypeStruct((NQT, 256), jnp.int32),
                   jax.ShapeDtypeStruct((NQT, 1024), jnp.float32)],
    )(query, rpx, rpy, qw, qb[None, :],
      jnp.asarray(_W_LANE), jnp.asarray(_H_LANE), jnp.asarray(_WP1_LANE),
      jnp.asarray(_HP1_LANE), jnp.asarray(_HL_LANE))
    return idx, wts


def _sc_gather_reduce(tables, idx, wts):
    mesh = plsc.VectorSubcoreMesh(core_axis_name="c", subcore_axis_name="s")

    @functools.partial(
        pl.kernel,
        out_type=jax.ShapeDtypeStruct((NQT, NH * DH), jnp.float32),
        mesh=mesh,
        scratch_types=[
            pltpu.VMEM((NH * PPR,), jnp.int32),
            pltpu.VMEM((4 * NH * PPR,), jnp.float32),
            pltpu.VMEM((NH, PPR, 128), jnp.float32),
            pltpu.VMEM((NH * DH,), jnp.float32),
            pltpu.SemaphoreType.DMA,
        ],
    )
    def body(t0, t1, t2, t3, idx_hbm, w_hbm, out_hbm,
             idx_v, w_v, rows_v, outb, sem):
        tbls = (t0, t1, t2, t3)
        wid = lax.axis_index("s") * 2 + lax.axis_index("c")
        qbase = wid * QPW
        splats = [jnp.full((LANES, 1), t, jnp.int32) for t in range(LANES)]
        gdn = lax.GatherDimensionNumbers(
            offset_dims=(), collapsed_slice_dims=(0,), start_index_map=(0,))

        def bcast(vec, t):
            return lax.gather(vec, splats[t], gdn, (1,),
                              mode=lax.GatherScatterMode.PROMISE_IN_BOUNDS)

        def blk_body(i, carry):
            qi = qbase + i
            pltpu.sync_copy(idx_hbm.at[qi], idx_v)
            pltpu.sync_copy(w_hbm.at[qi], w_v)
            copies = [
                pltpu.async_copy(
                    tbls[l].at[idx_v.at[pl.ds(h * PPR + l * NP, NP)]],
                    rows_v.at[h, pl.ds(l * NP, NP)], sem)
                for h in range(NH) for l in range(NL)
            ]
            for cp in copies:
                cp.wait()

            def row_body(r, carry2):
                z = jnp.zeros((LANES,), jnp.float32)
                a0, a1 = z, z
                for c in range(4):
                    for ch in range(2):
                        wchunk = w_v[pl.ds(c * 256 + r * PPR + ch * LANES,
                                           LANES)]
                        for t in range(LANES):
                            p = ch * LANES + t
                            wb = bcast(wchunk, t)
                            lo = rows_v[r, p, pl.ds(c * DH, LANES)]
                            hi = rows_v[r, p, pl.ds(c * DH + LANES, LANES)]
                            a0 = a0 + wb * lo
                            a1 = a1 + wb * hi
                outb[pl.ds(r * DH, LANES)] = a0
                outb[pl.ds(r * DH + LANES, LANES)] = a1
                return carry2

            lax.fori_loop(0, NH, row_body, 0)
            pltpu.sync_copy(outb, out_hbm.at[qi])
            return carry

        lax.fori_loop(0, QPW, blk_body, 0)

    return body(*tables, idx, wts)


def kernel(query, value, reference_points, spatial_shapes, level_start_index,
           W_off, b_off, W_attn, b_attn, W_val, b_val, W_out, b_out):
    bs, nq, d = query.shape
    nv = value.shape[1]

    v = _mm_bias(value.reshape(bs * nv, d), W_val, b_val)
    v3 = v.reshape(bs, nv, NH * DH)
    tables = []
    start = 0
    for (H, W) in _SS:
        vl = v3[:, start:start + H * W].reshape(bs, H, W, NH * DH)
        start += H * W
        tables.append(_quad_table(vl, H, W))

    wx = W_off.reshape(d, NH * NL * NP, 2)
    qw = jnp.concatenate([wx[:, :, 0], wx[:, :, 1], W_attn], axis=1)
    bx = b_off.reshape(NH * NL * NP, 2)
    qb = jnp.concatenate([bx[:, 0], bx[:, 1], b_attn], axis=0)
    rp = reference_points.reshape(NQT, 4, 2)
    idx, wts = _sampling_prep(query.reshape(NQT, d), rp[:, :, 0], rp[:, :, 1],
                              qw, qb)

    res = _sc_gather_reduce(tables, idx, wts)

    out = _mm_bias(res, W_out, b_out)
    return out.reshape(bs, nq, d)


# speedup vs baseline: 4.1535x; 1.9269x over previous
import functools

import numpy as np

import jax
import jax.numpy as jnp
from jax import lax
from jax.experimental import pallas as pl
from jax.experimental.pallas import tpu as pltpu
from jax.experimental.pallas import tpu_sc as plsc

EMBED = 256
NH = 8
NL = 4
NP = 8
DH = 32
_SS = ((92, 160), (46, 80), (23, 40), (12, 20))

BS = 2
NQ = 2048
NQT = BS * NQ
ROWS = NQT * NH
PPR = NL * NP
NWORK = 32
QPW = NQT // NWORK
QBLK = 256
LANES = 16

_lane = np.arange(NH * NL * NP)
_lane_l = (_lane // NP) % NL
_lane_h = _lane // (NL * NP)
_W_LANE = np.array([_SS[l][1] for l in _lane_l], np.float32)[None]
_H_LANE = np.array([_SS[l][0] for l in _lane_l], np.float32)[None]
_WP1_LANE = (_W_LANE + 1.0).astype(np.int32)
_HP1_LANE = (_H_LANE + 1.0).astype(np.int32)
_HL_LANE = _lane_h.astype(np.int32)[None]


def _mm_bias(x, w, b, block_m=512):
    M, K = x.shape
    N = w.shape[1]
    Mp = ((M + block_m - 1) // block_m) * block_m
    xp = jnp.pad(x, ((0, Mp - M), (0, 0))) if Mp != M else x

    def body(x_ref, w_ref, b_ref, o_ref):
        o_ref[...] = jnp.dot(x_ref[...], w_ref[...],
                             preferred_element_type=jnp.float32) + b_ref[...]

    out = pl.pallas_call(
        body,
        grid=(Mp // block_m,),
        in_specs=[pl.BlockSpec((block_m, K), lambda i: (i, 0)),
                  pl.BlockSpec((K, N), lambda i: (0, 0)),
                  pl.BlockSpec((1, N), lambda i: (0, 0))],
        out_specs=pl.BlockSpec((block_m, N), lambda i: (i, 0)),
        out_shape=jax.ShapeDtypeStruct((Mp, N), jnp.float32),
    )(xp, w, b[None, :])
    return out[:M]


def _quad_table(vl, H, W):
    bs = vl.shape[0]

    def body(top_ref, bot_ref, o_ref):
        a = pl.program_id(1)
        t = top_ref[0, 0] * jnp.where(a > 0, 1.0, 0.0)
        bo = bot_ref[0, 0] * jnp.where(a < H, 1.0, 0.0)
        z = jnp.zeros((1, NH * DH), jnp.float32)
        tdn = jnp.concatenate([z, t], axis=0)
        tup = jnp.concatenate([t, z], axis=0)
        bdn = jnp.concatenate([z, bo], axis=0)
        bup = jnp.concatenate([bo, z], axis=0)
        for h in range(NH):
            full = jnp.concatenate(
                [slot[:, h * DH:(h + 1) * DH]
                 for slot in (tdn, tup, bdn, bup)], axis=1)
            o_ref[0, 0, h] = full

    out = pl.pallas_call(
        body,
        grid=(bs, H + 1),
        in_specs=[
            pl.BlockSpec((1, 1, W, NH * DH),
                         lambda b, a: (b, jnp.maximum(a - 1, 0), 0, 0)),
            pl.BlockSpec((1, 1, W, NH * DH),
                         lambda b, a: (b, jnp.minimum(a, H - 1), 0, 0)),
        ],
        out_specs=pl.BlockSpec((1, 1, NH, W + 1, 4 * DH),
                               lambda b, a: (b, a, 0, 0, 0)),
        out_shape=jax.ShapeDtypeStruct((bs, H + 1, NH, W + 1, 4 * DH),
                                       jnp.float32),
    )(vl, vl)
    return out.reshape(bs * (H + 1) * NH * (W + 1), 4 * DH)


def _sampling_prep(query, rpx, rpy, qw, qb):
    def body(q_ref, rx_ref, ry_ref, w_ref, b_ref, wl_ref, hl_ref,
             wp1_ref, hp1_ref, hid_ref, idx_ref, wts_ref):
        i = pl.program_id(0)
        b = i // (NQ // QBLK)
        proj = jnp.dot(q_ref[...], w_ref[...],
                       preferred_element_type=jnp.float32) + b_ref[...]
        offx = proj[:, 0:256]
        offy = proj[:, 256:512]
        logit = proj[:, 512:768]
        segs = []
        for h in range(NH):
            seg = logit[:, h * 32:(h + 1) * 32]
            m = jnp.max(seg, axis=1, keepdims=True)
            e = jnp.exp(seg - m)
            segs.append(e / jnp.sum(e, axis=1, keepdims=True))
        aw = jnp.concatenate(segs, axis=1)

        rx = jnp.tile(rx_ref[...], (1, 64))
        ry = jnp.tile(ry_ref[...], (1, 64))
        Wl = wl_ref[...]
        Hl = hl_ref[...]
        x = rx * Wl + offx - 0.5
        y = ry * Hl + offy - 0.5
        x0 = jnp.floor(x)
        y0 = jnp.floor(y)
        tx = x - x0
        ty = y - y0
        vx0 = ((x0 >= 0.0) & (x0 < Wl)).astype(jnp.float32)
        vx1 = ((x0 + 1.0 >= 0.0) & (x0 + 1.0 < Wl)).astype(jnp.float32)
        vy0 = ((y0 >= 0.0) & (y0 < Hl)).astype(jnp.float32)
        vy1 = ((y0 + 1.0 >= 0.0) & (y0 + 1.0 < Hl)).astype(jnp.float32)
        xi = jnp.clip(x0, -1.0, Wl - 1.0).astype(jnp.int32)
        yi = jnp.clip(y0, -1.0, Hl - 1.0).astype(jnp.int32)
        pidx = ((b * hp1_ref[...] + (yi + 1)) * NH + hid_ref[...]) \
            * wp1_ref[...] + (xi + 1)
        idx_ref[...] = pidx
        wx0 = (1.0 - tx) * vx0
        wx1 = tx * vx1
        wy0 = (1.0 - ty) * vy0
        wy1 = ty * vy1
        for c, wc in enumerate((wx0 * wy0, wx1 * wy0, wx0 * wy1, wx1 * wy1)):
            wts_ref[:, pl.ds(c * 256, 256)] = wc * aw

    idx, wts = pl.pallas_call(
        body,
        grid=(NQT // QBLK,),
        in_specs=[pl.BlockSpec((QBLK, EMBED), lambda i: (i, 0)),
                  pl.BlockSpec((QBLK, 4), lambda i: (i, 0)),
                  pl.BlockSpec((QBLK, 4), lambda i: (i, 0)),
                  pl.BlockSpec((EMBED, 768), lambda i: (0, 0)),
                  pl.BlockSpec((1, 768), lambda i: (0, 0))]
        + [pl.BlockSpec((1, 256), lambda i: (0, 0))] * 5,
        out_specs=[pl.BlockSpec((QBLK, 256), lambda i: (i, 0)),
                   pl.BlockSpec((QBLK, 1024), lambda i: (i, 0))],
        out_shape=[jax.ShapeDtypeStruct((NQT, 256), jnp.int32),
                   jax.ShapeDtypeStruct((NQT, 1024), jnp.float32)],
    )(query, rpx, rpy, qw, qb[None, :],
      jnp.asarray(_W_LANE), jnp.asarray(_H_LANE), jnp.asarray(_WP1_LANE),
      jnp.asarray(_HP1_LANE), jnp.asarray(_HL_LANE))
    return idx, wts


def _sc_gather_reduce(tables, idx, wts):
    mesh = plsc.VectorSubcoreMesh(core_axis_name="c", subcore_axis_name="s")

    @functools.partial(
        pl.kernel,
        out_type=jax.ShapeDtypeStruct((NQT, NH * DH), jnp.float32),
        mesh=mesh,
        scratch_types=[
            pltpu.VMEM((2, NH * PPR), jnp.int32),
            pltpu.VMEM((2, 4 * NH * PPR), jnp.float32),
            pltpu.VMEM((2, NH, PPR, 128), jnp.float32),
            pltpu.VMEM((NH * DH,), jnp.float32),
            pltpu.SemaphoreType.DMA,
            pltpu.SemaphoreType.DMA,
            pltpu.SemaphoreType.DMA,
            pltpu.SemaphoreType.DMA,
        ],
    )
    def body(t0, t1, t2, t3, idx_hbm, w_hbm, out_hbm,
             idx_v, w_v, rows_v, outb, si0, si1, sg0, sg1):
        tbls = (t0, t1, t2, t3)
        si = (si0, si1)
        sg = (sg0, sg1)
        wid = lax.axis_index("s") * 2 + lax.axis_index("c")
        qbase = wid * QPW
        splats = [jnp.full((LANES, 1), t, jnp.int32) for t in range(LANES)]
        gdn = lax.GatherDimensionNumbers(
            offset_dims=(), collapsed_slice_dims=(0,), start_index_map=(0,))

        def bcast(vec, t):
            return lax.gather(vec, splats[t], gdn, (1,),
                              mode=lax.GatherScatterMode.PROMISE_IN_BOUNDS)

        def start_idxw(j, p):
            pltpu.async_copy(idx_hbm.at[qbase + j], idx_v.at[p], si[p])
            pltpu.async_copy(w_hbm.at[qbase + j], w_v.at[p], si[p])

        def drain_idxw(p):
            pltpu.make_async_copy(idx_hbm.at[qbase], idx_v.at[p], si[p]).wait()
            pltpu.make_async_copy(w_hbm.at[qbase], w_v.at[p], si[p]).wait()

        def start_gathers(p):
            for h in range(NH):
                for l in range(NL):
                    pltpu.async_copy(
                        tbls[l].at[idx_v.at[p, pl.ds(h * PPR + l * NP, NP)]],
                        rows_v.at[p, h, pl.ds(l * NP, NP)], sg[p])

        def drain_gathers(p):
            for h in range(NH):
                for l in range(NL):
                    pltpu.make_async_copy(
                        tbls[0].at[idx_v.at[p, pl.ds(h * PPR + l * NP, NP)]],
                        rows_v.at[p, h, pl.ds(l * NP, NP)], sg[p]).wait()

        def compute(p, qi):
            def row_body(r, carry2):
                z = jnp.zeros((LANES,), jnp.float32)
                a0, a1 = z, z
                for c in range(4):
                    for ch in range(2):
                        wchunk = w_v[p, pl.ds(c * 256 + r * PPR + ch * LANES,
                                              LANES)]
                        for t in range(LANES):
                            pt = ch * LANES + t
                            wb = bcast(wchunk, t)
                            lo = rows_v[p, r, pt, pl.ds(c * DH, LANES)]
                            hi = rows_v[p, r, pt, pl.ds(c * DH + LANES, LANES)]
                            a0 = a0 + wb * lo
                            a1 = a1 + wb * hi
                outb[pl.ds(r * DH, LANES)] = a0
                outb[pl.ds(r * DH + LANES, LANES)] = a1
                return carry2

            lax.fori_loop(0, NH, row_body, 0)
            pltpu.sync_copy(outb, out_hbm.at[qi])

        start_idxw(0, 0)
        drain_idxw(0)
        start_gathers(0)
        start_idxw(1, 1)

        def two_blocks(j2, carry):
            j = j2 * 2
            for p in (0, 1):
                jj = j + p

                @pl.when(jj + 1 < QPW)
                def _():
                    drain_idxw(1 - p)
                    start_gathers(1 - p)

                drain_gathers(p)
                compute(p, qbase + jj)

                @pl.when(jj + 2 < QPW)
                def _():
                    start_idxw(jj + 2, p)
            return carry

        lax.fori_loop(0, QPW // 2, two_blocks, 0)

    return body(*tables, idx, wts)


def kernel(query, value, reference_points, spatial_shapes, level_start_index,
           W_off, b_off, W_attn, b_attn, W_val, b_val, W_out, b_out):
    bs, nq, d = query.shape
    nv = value.shape[1]

    v = _mm_bias(value.reshape(bs * nv, d), W_val, b_val)
    v3 = v.reshape(bs, nv, NH * DH)
    tables = []
    start = 0
    for (H, W) in _SS:
        vl = v3[:, start:start + H * W].reshape(bs, H, W, NH * DH)
        start += H * W
        tables.append(_quad_table(vl, H, W))

    wx = W_off.reshape(d, NH * NL * NP, 2)
    qw = jnp.concatenate([wx[:, :, 0], wx[:, :, 1], W_attn], axis=1)
    bx = b_off.reshape(NH * NL * NP, 2)
    qb = jnp.concatenate([bx[:, 0], bx[:, 1], b_attn], axis=0)
    rp = reference_points.reshape(NQT, 4, 2)
    idx, wts = _sampling_prep(query.reshape(NQT, d), rp[:, :, 0], rp[:, :, 1],
                              qw, qb)

    res = _sc_gather_reduce(tables, idx, wts)

    out = _mm_bias(res, W_out, b_out)
    return out.reshape(bs, nq, d)


# speedup vs baseline: 4.9255x; 1.1859x over previous
import functools

import numpy as np

import jax
import jax.numpy as jnp
from jax import lax
from jax.experimental import pallas as pl
from jax.experimental.pallas import tpu as pltpu
from jax.experimental.pallas import tpu_sc as plsc

EMBED = 256
NH = 8
NL = 4
NP = 8
DH = 32
_SS = ((92, 160), (46, 80), (23, 40), (12, 20))

BS = 2
NQ = 2048
NQT = BS * NQ
ROWS = NQT * NH
PPR = NL * NP
NWORK = 32
QPW = NQT // NWORK
QBLK = 256
LANES = 16

_lane = np.arange(NH * NL * NP)
_lane_l = (_lane // NP) % NL
_lane_h = _lane // (NL * NP)
_W_LANE = np.array([_SS[l][1] for l in _lane_l], np.float32)[None]
_H_LANE = np.array([_SS[l][0] for l in _lane_l], np.float32)[None]
_WS = tuple(-(-(w + 1) // 8) * 8 for _, w in _SS)
_WS_LANE = np.array([_WS[l] for l in _lane_l], np.int32)[None]
_HP1_LANE = (_H_LANE + 1.0).astype(np.int32)
_HL_LANE = _lane_h.astype(np.int32)[None]


def _mm_bias(x, w, b, block_m=512):
    M, K = x.shape
    N = w.shape[1]
    Mp = ((M + block_m - 1) // block_m) * block_m
    xp = jnp.pad(x, ((0, Mp - M), (0, 0))) if Mp != M else x

    def body(x_ref, w_ref, b_ref, o_ref):
        o_ref[...] = jnp.dot(x_ref[...], w_ref[...],
                             preferred_element_type=jnp.float32) + b_ref[...]

    out = pl.pallas_call(
        body,
        grid=(Mp // block_m,),
        in_specs=[pl.BlockSpec((block_m, K), lambda i: (i, 0)),
                  pl.BlockSpec((K, N), lambda i: (0, 0)),
                  pl.BlockSpec((1, N), lambda i: (0, 0))],
        out_specs=pl.BlockSpec((block_m, N), lambda i: (i, 0)),
        out_shape=jax.ShapeDtypeStruct((Mp, N), jnp.float32),
    )(xp, w, b[None, :])
    return out[:M]


def _quad_table(vl, H, W):
    bs = vl.shape[0]

    def body(top_ref, bot_ref, o_ref):
        a = pl.program_id(1)
        t = top_ref[0, 0] * jnp.where(a > 0, 1.0, 0.0)
        bo = bot_ref[0, 0] * jnp.where(a < H, 1.0, 0.0)
        z = jnp.zeros((1, NH * DH), jnp.float32)
        tdn = jnp.concatenate([z, t], axis=0)
        tup = jnp.concatenate([t, z], axis=0)
        bdn = jnp.concatenate([z, bo], axis=0)
        bup = jnp.concatenate([bo, z], axis=0)
        for h in range(NH):
            full = jnp.concatenate(
                [slot[:, h * DH:(h + 1) * DH]
                 for slot in (tdn, tup, bdn, bup)], axis=1)
            o_ref[0, 0, h, pl.ds(0, W + 1)] = full

    WS = -(-(W + 1) // 8) * 8
    out = pl.pallas_call(
        body,
        grid=(bs, H + 1),
        in_specs=[
            pl.BlockSpec((1, 1, W, NH * DH),
                         lambda b, a: (b, jnp.maximum(a - 1, 0), 0, 0)),
            pl.BlockSpec((1, 1, W, NH * DH),
                         lambda b, a: (b, jnp.minimum(a, H - 1), 0, 0)),
        ],
        out_specs=pl.BlockSpec((1, 1, NH, WS, 4 * DH),
                               lambda b, a: (b, a, 0, 0, 0)),
        out_shape=jax.ShapeDtypeStruct((bs, H + 1, NH, WS, 4 * DH),
                                       jnp.float32),
    )(vl, vl)
    return out.reshape(bs * (H + 1) * NH * WS, 4 * DH)


def _sampling_prep(query, rpx, rpy, qw, qb):
    def body(q_ref, rx_ref, ry_ref, w_ref, b_ref, wl_ref, hl_ref,
             ws_ref, hp1_ref, hid_ref, idx_ref, wts_ref):
        i = pl.program_id(0)
        b = i // (NQ // QBLK)
        proj = jnp.dot(q_ref[...], w_ref[...],
                       preferred_element_type=jnp.float32) + b_ref[...]
        offx = proj[:, 0:256]
        offy = proj[:, 256:512]
        logit = proj[:, 512:768]
        segs = []
        for h in range(NH):
            seg = logit[:, h * 32:(h + 1) * 32]
            m = jnp.max(seg, axis=1, keepdims=True)
            e = jnp.exp(seg - m)
            segs.append(e / jnp.sum(e, axis=1, keepdims=True))
        aw = jnp.concatenate(segs, axis=1)

        rx = jnp.tile(rx_ref[...], (1, 64))
        ry = jnp.tile(ry_ref[...], (1, 64))
        Wl = wl_ref[...]
        Hl = hl_ref[...]
        x = rx * Wl + offx - 0.5
        y = ry * Hl + offy - 0.5
        x0 = jnp.floor(x)
        y0 = jnp.floor(y)
        tx = x - x0
        ty = y - y0
        vx0 = ((x0 >= 0.0) & (x0 < Wl)).astype(jnp.float32)
        vx1 = ((x0 + 1.0 >= 0.0) & (x0 + 1.0 < Wl)).astype(jnp.float32)
        vy0 = ((y0 >= 0.0) & (y0 < Hl)).astype(jnp.float32)
        vy1 = ((y0 + 1.0 >= 0.0) & (y0 + 1.0 < Hl)).astype(jnp.float32)
        xi = jnp.clip(x0, -1.0, Wl - 1.0).astype(jnp.int32)
        yi = jnp.clip(y0, -1.0, Hl - 1.0).astype(jnp.int32)
        pidx = ((b * hp1_ref[...] + (yi + 1)) * NH + hid_ref[...]) \
            * ws_ref[...] + (xi + 1)
        idx_ref[...] = pidx
        wx0 = (1.0 - tx) * vx0
        wx1 = tx * vx1
        wy0 = (1.0 - ty) * vy0
        wy1 = ty * vy1
        for c, wc in enumerate((wx0 * wy0, wx1 * wy0, wx0 * wy1, wx1 * wy1)):
            wts_ref[:, pl.ds(c * 256, 256)] = wc * aw

    idx, wts = pl.pallas_call(
        body,
        grid=(NQT // QBLK,),
        in_specs=[pl.BlockSpec((QBLK, EMBED), lambda i: (i, 0)),
                  pl.BlockSpec((QBLK, 4), lambda i: (i, 0)),
                  pl.BlockSpec((QBLK, 4), lambda i: (i, 0)),
                  pl.BlockSpec((EMBED, 768), lambda i: (0, 0)),
                  pl.BlockSpec((1, 768), lambda i: (0, 0))]
        + [pl.BlockSpec((1, 256), lambda i: (0, 0))] * 5,
        out_specs=[pl.BlockSpec((QBLK, 256), lambda i: (i, 0)),
                   pl.BlockSpec((QBLK, 1024), lambda i: (i, 0))],
        out_shape=[jax.ShapeDtypeStruct((NQT, 256), jnp.int32),
                   jax.ShapeDtypeStruct((NQT, 1024), jnp.float32)],
    )(query, rpx, rpy, qw, qb[None, :],
      jnp.asarray(_W_LANE), jnp.asarray(_H_LANE), jnp.asarray(_WS_LANE),
      jnp.asarray(_HP1_LANE), jnp.asarray(_HL_LANE))
    return idx, wts


def _sc_gather_reduce(tables, idx, wts):
    mesh = plsc.VectorSubcoreMesh(core_axis_name="c", subcore_axis_name="s")

    @functools.partial(
        pl.kernel,
        out_type=jax.ShapeDtypeStruct((NQT, NH * DH), jnp.float32),
        mesh=mesh,
        scratch_types=[
            pltpu.VMEM((2, NH * PPR), jnp.int32),
            pltpu.VMEM((2, 4 * NH * PPR), jnp.float32),
            pltpu.VMEM((2, NH, PPR, 128), jnp.float32),
            pltpu.VMEM((NH * DH,), jnp.float32),
            pltpu.SemaphoreType.DMA,
            pltpu.SemaphoreType.DMA,
            pltpu.SemaphoreType.DMA,
            pltpu.SemaphoreType.DMA,
        ],
    )
    def body(t0, t1, t2, t3, idx_hbm, w_hbm, out_hbm,
             idx_v, w_v, rows_v, outb, si0, si1, sg0, sg1):
        tbls = (t0, t1, t2, t3)
        si = (si0, si1)
        sg = (sg0, sg1)
        wid = lax.axis_index("s") * 2 + lax.axis_index("c")
        qbase = wid * QPW
        splats = [jnp.full((LANES, 1), t, jnp.int32) for t in range(LANES)]
        gdn = lax.GatherDimensionNumbers(
            offset_dims=(), collapsed_slice_dims=(0,), start_index_map=(0,))

        def bcast(vec, t):
            return lax.gather(vec, splats[t], gdn, (1,),
                              mode=lax.GatherScatterMode.PROMISE_IN_BOUNDS)

        def start_idxw(j, p):
            pltpu.async_copy(idx_hbm.at[qbase + j], idx_v.at[p], si[p])
            pltpu.async_copy(w_hbm.at[qbase + j], w_v.at[p], si[p])

        def drain_idxw(p):
            pltpu.make_async_copy(idx_hbm.at[qbase], idx_v.at[p], si[p]).wait()
            pltpu.make_async_copy(w_hbm.at[qbase], w_v.at[p], si[p]).wait()

        def start_gathers(p):
            for h in range(NH):
                for l in range(NL):
                    pltpu.async_copy(
                        tbls[l].at[idx_v.at[p, pl.ds(h * PPR + l * NP, NP)]],
                        rows_v.at[p, h, pl.ds(l * NP, NP)], sg[p])

        def drain_gathers(p):
            for h in range(NH):
                for l in range(NL):
                    pltpu.make_async_copy(
                        tbls[0].at[idx_v.at[p, pl.ds(h * PPR + l * NP, NP)]],
                        rows_v.at[p, h, pl.ds(l * NP, NP)], sg[p]).wait()

        def compute(p, qi):
            def row_body(r, carry2):
                z = jnp.zeros((LANES,), jnp.float32)
                a0, a1 = z, z
                for c in range(4):
                    for ch in range(2):
                        wchunk = w_v[p, pl.ds(c * 256 + r * PPR + ch * LANES,
                                              LANES)]
                        for t in range(LANES):
                            pt = ch * LANES + t
                            wb = bcast(wchunk, t)
                            lo = rows_v[p, r, pt, pl.ds(c * DH, LANES)]
                            hi = rows_v[p, r, pt, pl.ds(c * DH + LANES, LANES)]
                            a0 = a0 + wb * lo
                            a1 = a1 + wb * hi
                outb[pl.ds(r * DH, LANES)] = a0
                outb[pl.ds(r * DH + LANES, LANES)] = a1
                return carry2

            lax.fori_loop(0, NH, row_body, 0)
            pltpu.sync_copy(outb, out_hbm.at[qi])

        start_idxw(0, 0)
        drain_idxw(0)
        start_gathers(0)
        start_idxw(1, 1)

        def two_blocks(j2, carry):
            j = j2 * 2
            for p in (0, 1):
                jj = j + p

                @pl.when(jj + 1 < QPW)
                def _():
                    drain_idxw(1 - p)
                    start_gathers(1 - p)

                drain_gathers(p)
                compute(p, qbase + jj)

                @pl.when(jj + 2 < QPW)
                def _():
                    start_idxw(jj + 2, p)
            return carry

        lax.fori_loop(0, QPW // 2, two_blocks, 0)

    return body(*tables, idx, wts)


def kernel(query, value, reference_points, spatial_shapes, level_start_index,
           W_off, b_off, W_attn, b_attn, W_val, b_val, W_out, b_out):
    bs, nq, d = query.shape
    nv = value.shape[1]

    v = _mm_bias(value.reshape(bs * nv, d), W_val, b_val, block_m=240)
    v3 = v.reshape(bs, nv, NH * DH)
    tables = []
    start = 0
    for (H, W) in _SS:
        vl = v3[:, start:start + H * W].reshape(bs, H, W, NH * DH)
        start += H * W
        tables.append(_quad_table(vl, H, W))

    wx = W_off.reshape(d, NH * NL * NP, 2)
    qw = jnp.concatenate([wx[:, :, 0], wx[:, :, 1], W_attn], axis=1)
    bx = b_off.reshape(NH * NL * NP, 2)
    qb = jnp.concatenate([bx[:, 0], bx[:, 1], b_attn], axis=0)
    rp = reference_points.reshape(NQT, 4, 2)
    idx, wts = _sampling_prep(query.reshape(NQT, d), rp[:, :, 0], rp[:, :, 1],
                              qw, qb)

    res = _sc_gather_reduce(tables, idx, wts)

    out = _mm_bias(res, W_out, b_out)
    return out.reshape(bs, nq, d)
